# baseline (speedup 1.0000x reference)
import functools

import jax
import jax.numpy as jnp
from jax import lax
from jax.experimental import pallas as pl
from jax.experimental.pallas import tpu as pltpu
from jax.experimental.pallas import tpu_sc as plsc

_NC = 2
_NS = 16
_NW = _NC * _NS
_NBUF = 4


@functools.partial(jax.jit, static_argnames=("b", "b_per_w", "chunk"))
def _sc_gather(x, idx, *, b, b_per_w, chunk):
    sizes = [chunk] * (b_per_w // chunk)
    if b_per_w % chunk:
        sizes.append(b_per_w % chunk)
    offs = [sum(sizes[:g]) for g in range(len(sizes))]
    n_chunks = len(sizes)
    d = x.shape[1]
    last_span = b - (_NW - 1) * b_per_w
    last_len = [min(max(last_span - offs[g], 0), sizes[g]) for g in range(n_chunks)]
    assert 0 < last_span <= b_per_w and all(l % 8 == 0 for l in last_len)
    mesh = plsc.VectorSubcoreMesh(core_axis_name="c", subcore_axis_name="s")

    @functools.partial(
        pl.kernel,
        mesh=mesh,
        out_type=jax.ShapeDtypeStruct((b, d), jnp.float32),
        scratch_types=[
            pltpu.VMEM((b_per_w,), jnp.int32),
            pltpu.VMEM((_NBUF, chunk, d), jnp.float32),
        ]
        + [pltpu.SemaphoreType.DMA] * _NBUF,
    )
    def k(table_hbm, idx_hbm, out_hbm, idx_v, bufs, *sems):
        wid = lax.axis_index("s") * _NC + lax.axis_index("c")
        base = wid * b_per_w
        pltpu.sync_copy(idx_hbm.at[pl.ds(base, b_per_w)], idx_v)

        def start_gather(g):
            return pltpu.async_copy(
                table_hbm.at[idx_v.at[pl.ds(offs[g], sizes[g])]],
                bufs.at[g % _NBUF].at[pl.ds(0, sizes[g])],
                sems[g % _NBUF],
            )

        copies = [None] * _NBUF
        for g in range(min(_NBUF, n_chunks)):
            copies[g] = start_gather(g)
        for g in range(n_chunks):
            cur = g % _NBUF
            copies[cur].wait()
            if last_len[g] == sizes[g]:
                pltpu.sync_copy(
                    bufs.at[cur].at[pl.ds(0, sizes[g])],
                    out_hbm.at[pl.ds(base + offs[g], sizes[g])],
                )
            else:
                full = base + b_per_w <= b

                @pl.when(full)
                def _():
                    pltpu.sync_copy(
                        bufs.at[cur].at[pl.ds(0, sizes[g])],
                        out_hbm.at[pl.ds(base + offs[g], sizes[g])],
                    )

                if last_len[g] > 0:
                    lw = last_len[g]

                    @pl.when(jnp.logical_not(full))
                    def _():
                        pltpu.sync_copy(
                            bufs.at[cur].at[pl.ds(0, lw)],
                            out_hbm.at[pl.ds(base + offs[g], lw)],
                        )
            if g + _NBUF < n_chunks:
                copies[cur] = start_gather(g + _NBUF)

    return k(x, idx)


def kernel(x, coarse_idx):
    b = coarse_idx.shape[0]
    b_per_w = -(-b // (_NW * 8)) * 8
    idx = jnp.zeros((b_per_w * _NW,), jnp.int32).at[:b].set(
        coarse_idx.astype(jnp.int32))
    return _sc_gather(x, idx, b=b, b_per_w=b_per_w, chunk=112)
